# baseline (speedup 1.0000x reference)
import functools

import jax
import jax.numpy as jnp
from jax import lax
from jax.experimental import pallas as pl
from jax.experimental.pallas import tpu as pltpu
from jax.experimental.pallas import tpu_sc as plsc

_BATCH = 16384
_FEAT = 64
_LANES = 16

_NC = 2
_NS = 16
_NW = _NC * _NS
_ROWS_W = _BATCH // _NW
_IDX_CHUNK = 128
_N_CHUNKS = _ROWS_W // _IDX_CHUNK
_VECS_PER_ROW = _FEAT // _LANES


@functools.partial(
    pl.kernel,
    mesh=plsc.VectorSubcoreMesh(core_axis_name="c", subcore_axis_name="s"),
    compiler_params=pltpu.CompilerParams(use_tc_tiling_on_sc=False),
    out_type=jax.ShapeDtypeStruct((_NW, _LANES), jnp.float32),
    scratch_types=[
        pltpu.VMEM((_N_CHUNKS, _IDX_CHUNK), jnp.int32),
        pltpu.VMEM((_ROWS_W, _FEAT), jnp.float32),
        pltpu.VMEM((_ROWS_W, _FEAT), jnp.float32),
        pltpu.VMEM((_LANES,), jnp.float32),
        pltpu.SemaphoreType.DMA,
        pltpu.SemaphoreType.DMA,
    ],
)
def _center_loss_partials(x_hbm, labels_hbm, centers_hbm, out_hbm,
                          idx_v, rows_v, x_v, acc_v, sem_x, sem_g):
    wid = lax.axis_index("s") * _NC + lax.axis_index("c")

    x_copy = pltpu.async_copy(x_hbm.at[wid], x_v, sem_x)

    pltpu.sync_copy(labels_hbm.at[wid], idx_v)
    gathers = []
    for j in range(_N_CHUNKS):
        gathers.append(
            pltpu.async_copy(
                centers_hbm.at[idx_v.at[j]],
                rows_v.at[pl.ds(j * _IDX_CHUNK, _IDX_CHUNK)],
                sem_g,
            )
        )
    for g in gathers:
        g.wait()
    x_copy.wait()

    def row_body(r, acc):
        for cth in range(_VECS_PER_ROW):
            xv = x_v[r, pl.ds(cth * _LANES, _LANES)]
            cv = rows_v[r, pl.ds(cth * _LANES, _LANES)]
            d = xv - cv
            acc = acc + d * d
        return acc

    acc = lax.fori_loop(0, _ROWS_W, row_body,
                        jnp.zeros((_LANES,), jnp.float32))

    acc_v[...] = acc
    pltpu.sync_copy(acc_v, out_hbm.at[wid])


def kernel(x, labels, centers):
    x_r = x.reshape(_NW, _ROWS_W, _FEAT)
    labels_r = labels.astype(jnp.int32).reshape(_NW, _N_CHUNKS, _IDX_CHUNK)
    partials = _center_loss_partials(x_r, labels_r, centers)
    return jnp.sum(partials) * (1.0 / _BATCH)
